# baseline (speedup 1.0000x reference)
import functools

import jax
import jax.numpy as jnp
from jax import lax
from jax.experimental import pallas as pl
from jax.experimental.pallas import tpu as pltpu
from jax.experimental.pallas import tpu_sc as plsc

N = 10000
E = 320000
D_IN = 128
H = 64

NC = 2
NS = 16
NW = NC * NS
R = 320
NPAD = NW * R
SEG = 40000
NSEG = E // SEG
CAP = SEG + 16
CH = 128
GB = CH // 2
LTOT = NW * NSEG * SEG + CH
GCAP_CH = E // CH + NW * NSEG + 4
GROWS = GCAP_CH * GB

_mesh = plsc.VectorSubcoreMesh(core_axis_name="c", subcore_axis_name="s")
_sc_params = pltpu.CompilerParams(needs_layout_passes=False)


def _wid():
    return lax.axis_index("s") * NC + lax.axis_index("c")


@functools.partial(
    pl.kernel,
    out_type=(
        jax.ShapeDtypeStruct((LTOT,), jnp.int32),
        jax.ShapeDtypeStruct((NW * NSEG * 16,), jnp.int32),
        jax.ShapeDtypeStruct((NW * 16,), jnp.int32),
    ),
    mesh=_mesh,
    compiler_params=_sc_params,
    scratch_types=[
        pltpu.VMEM((SEG,), jnp.int32),
        pltpu.VMEM((SEG,), jnp.int32),
        pltpu.VMEM((CAP,), jnp.int32),
        pltpu.VMEM((NSEG * 16,), jnp.int32),
        pltpu.VMEM((16,), jnp.int32),
    ],
)
def _bin_kernel(dst_hbm, src_hbm, lists_hbm, cnts_hbm, tots_hbm,
                dbuf, sbuf, cbuf, cntv, tbuf):
    t = _wid()
    lo = t * R

    def zero_body(i, c):
        cbuf[pl.ds(i * 16, 16)] = jnp.zeros((16,), jnp.int32)
        return c

    lax.fori_loop(0, CAP // 16, zero_body, 0)

    def seg_body(s, tot):
        pltpu.sync_copy(dst_hbm.at[pl.ds(s * SEG, SEG)], dbuf)
        pltpu.sync_copy(src_hbm.at[pl.ds(s * SEG, SEG)], sbuf)

        def inner(i, cur):
            d = dbuf[pl.ds(i * 16, 16)]
            sv = sbuf[pl.ds(i * 16, 16)]
            dl = d - lo
            m = (dl >= 0) & (dl < R)
            packed = (sv << 9) | (dl & 511)
            csum = plsc.cumsum(m.astype(jnp.int32))
            plsc.store_scatter(cbuf, [cur + csum - 1], packed, mask=m)
            return cur + csum[15]

        cur = lax.fori_loop(0, SEG // 16, inner, 0)
        cntv[pl.ds(s * 16, 16)] = jnp.broadcast_to(cur, (16,))
        pltpu.sync_copy(cbuf.at[pl.ds(0, SEG)],
                        lists_hbm.at[pl.ds((t * NSEG + s) * SEG, SEG)])
        return tot + (cur + CH - 1) // CH

    tot = lax.fori_loop(0, NSEG, seg_body, 0)
    tbuf[pl.ds(0, 16)] = jnp.broadcast_to(tot, (16,))
    pltpu.sync_copy(cntv, cnts_hbm.at[pl.ds(t * NSEG * 16, NSEG * 16)])
    pltpu.sync_copy(tbuf, tots_hbm.at[pl.ds(t * 16, 16)])


def _chunk_base(t, totv):
    def pre(u, eb):
        return eb + totv[pl.ds(u * 16, 16)][0]

    return lax.fori_loop(0, t, pre, 0)


@functools.partial(
    pl.kernel,
    out_type=jax.ShapeDtypeStruct((GROWS, 2 * H), jnp.float32),
    mesh=_mesh,
    compiler_params=_sc_params,
    scratch_types=[
        pltpu.VMEM((CH + 16,), jnp.int32),
        pltpu.VMEM((CH,), jnp.int32),
        pltpu.VMEM((CH,), jnp.int32),
        pltpu.VMEM((CH, 2 * H), jnp.float32),
        pltpu.VMEM((CH, 2 * H), jnp.float32),
        pltpu.VMEM((GB, 2 * H), jnp.float32),
        pltpu.VMEM((NSEG * 16,), jnp.int32),
        pltpu.VMEM((NW * 16,), jnp.int32),
        pltpu.SemaphoreType.DMA,
        pltpu.SemaphoreType.DMA,
    ],
)
def _gather_kernel(tab_hbm, lists_hbm, cnts_hbm, tots_hbm, g_hbm,
                   lbuf, didx, sidx, rd, rs, gbuf, cntv, totv, sa, sb):
    t = _wid()
    lo = t * R
    iota = lax.iota(jnp.int32, 16)
    pltpu.sync_copy(cnts_hbm.at[pl.ds(t * NSEG * 16, NSEG * 16)], cntv)
    pltpu.sync_copy(tots_hbm, totv)
    ob0 = _chunk_base(t, totv)

    def chunk(loff, ne, ob):
        pltpu.sync_copy(lists_hbm.at[pl.ds(loff, CH)], lbuf.at[pl.ds(0, CH)])

        def up(k, u):
            v = lbuf[pl.ds(k * 16, 16)]
            pos = k * 16 + iota
            valid = pos < ne
            sidx[pl.ds(k * 16, 16)] = jnp.where(
                valid, lax.shift_right_logical(v, 9), pos)
            didx[pl.ds(k * 16, 16)] = jnp.where(valid, lo + (v & 511), pos)
            return u

        lax.fori_loop(0, CH // 16, up, 0)
        cpa = pltpu.async_copy(tab_hbm.at[didx], rd, sa)
        cpb = pltpu.async_copy(tab_hbm.at[sidx], rs, sb)
        cpa.wait()
        cpb.wait()

        def add_body(q, c):
            for half in range(2):
                r = 2 * q + half
                for k in range(H // 16):
                    a = rd[r, pl.ds(k * 16, 16)]
                    b = rs[r, pl.ds(H + k * 16, 16)]
                    gbuf[q, pl.ds(half * H + k * 16, 16)] = a + b
            return c

        lax.fori_loop(0, GB, add_body, 0)
        pltpu.sync_copy(gbuf,
                        g_hbm.at[pl.ds(pl.multiple_of(ob * GB, 8), GB)])

    def seg_body(s, ob):
        cnt = cntv[pl.ds(s * 16, 16)][0]
        nch = (cnt + CH - 1) // CH
        lbase = (t * NSEG + s) * SEG

        def ch_body(j, ob2):
            chunk(lbase + j * CH, cnt - j * CH, ob2)
            return ob2 + 1

        return lax.fori_loop(0, nch, ch_body, ob)

    lax.fori_loop(0, NSEG, seg_body, ob0)


@functools.partial(
    pl.kernel,
    out_type=jax.ShapeDtypeStruct((NPAD // 2, 2 * H), jnp.float32),
    mesh=_mesh,
    compiler_params=_sc_params,
    scratch_types=[
        pltpu.VMEM((R // 2, 2 * H), jnp.float32),
        pltpu.VMEM((CH + 16,), jnp.int32),
        pltpu.VMEM((GB, 2 * H), jnp.float32),
        pltpu.VMEM((NSEG * 16,), jnp.int32),
        pltpu.VMEM((NW * 16,), jnp.int32),
    ],
)
def _scatter_kernel(m_hbm, lists_hbm, cnts_hbm, tots_hbm, out_hbm,
                    acc, lbuf, rowbuf, cntv, totv):
    t = _wid()
    pltpu.sync_copy(cnts_hbm.at[pl.ds(t * NSEG * 16, NSEG * 16)], cntv)
    pltpu.sync_copy(tots_hbm, totv)
    ob0 = _chunk_base(t, totv)

    def zb(r, c):
        for k in range(2 * H // 16):
            acc[r, pl.ds(k * 16, 16)] = jnp.zeros((16,), jnp.float32)
        return c

    lax.fori_loop(0, R // 2, zb, 0)

    def chunk(loff, ne, ob):
        pltpu.sync_copy(lists_hbm.at[pl.ds(loff, CH)], lbuf.at[pl.ds(0, CH)])
        pltpu.sync_copy(m_hbm.at[pl.ds(pl.multiple_of(ob * GB, 8), GB)],
                        rowbuf)

        def e_body(e, ec):
            p = lbuf[pl.ds(e, 16)][0]
            dl = lax.bitwise_and(p, 511)
            ao = lax.bitwise_and(dl, 1) * H
            ar = lax.shift_right_logical(dl, 1)
            mo = lax.bitwise_and(e, 1) * H
            er = lax.shift_right_logical(e, 1)
            for k in range(H // 16):
                a = acc[ar, pl.ds(ao + k * 16, 16)]
                r = rowbuf[er, pl.ds(mo + k * 16, 16)]
                acc[ar, pl.ds(ao + k * 16, 16)] = jnp.maximum(a, r)
            return ec

        lax.fori_loop(0, ne, e_body, 0)

    def seg_body(s, ob):
        cnt = cntv[pl.ds(s * 16, 16)][0]
        nch = (cnt + CH - 1) // CH
        lbase = (t * NSEG + s) * SEG

        def ch_body(j, ob2):
            chunk(lbase + j * CH, jnp.minimum(CH, cnt - j * CH), ob2)
            return ob2 + 1

        return lax.fori_loop(0, nch, ch_body, ob)

    lax.fori_loop(0, NSEG, seg_body, ob0)
    pltpu.sync_copy(acc, out_hbm.at[pl.ds(pl.multiple_of(t * (R // 2), 8), R // 2)])


def _enc_body(x_ref, w1_ref, b1_ref, w2_ref, b2_ref, wab_ref, bab_ref, tab_ref):
    x = x_ref[...]
    h = jnp.maximum(jnp.dot(x, w1_ref[...], preferred_element_type=jnp.float32) + b1_ref[...], 0.0)
    h = jnp.dot(h, w2_ref[...], preferred_element_type=jnp.float32) + b2_ref[...]
    tab_ref[...] = jnp.dot(h, wab_ref[...], preferred_element_type=jnp.float32) + bab_ref[...]


def _ab_body(h_ref, wab_ref, bab_ref, tab_ref):
    tab_ref[...] = jnp.dot(h_ref[...], wab_ref[...], preferred_element_type=jnp.float32) + bab_ref[...]


def _msg_body(gp_ref, w2_ref, b2_ref, out_ref):
    g = jnp.maximum(gp_ref[...], 0.0)
    out_ref[...] = jnp.dot(g, w2_ref[...], preferred_element_type=jnp.float32) + b2_ref[...]


def _full(shape):
    return pl.BlockSpec(shape, lambda i: (0, 0))


_NBLK = 2000


def _enc_call(x, w1, b1, w2, b2, wab, bab):
    return pl.pallas_call(
        _enc_body,
        grid=(N // _NBLK,),
        in_specs=[
            pl.BlockSpec((_NBLK, D_IN), lambda i: (i, 0)),
            _full((D_IN, H)),
            _full((1, H)),
            _full((H, H)),
            _full((1, H)),
            _full((H, 2 * H)),
            _full((1, 2 * H)),
        ],
        out_specs=pl.BlockSpec((_NBLK, 2 * H), lambda i: (i, 0)),
        out_shape=jax.ShapeDtypeStruct((N, 2 * H), jnp.float32),
    )(x, w1, b1, w2, b2, wab, bab)


def _ab_call(h, wab, bab):
    return pl.pallas_call(
        _ab_body,
        grid=(N // _NBLK,),
        in_specs=[
            pl.BlockSpec((_NBLK, H), lambda i: (i, 0)),
            _full((H, 2 * H)),
            _full((1, 2 * H)),
        ],
        out_specs=pl.BlockSpec((_NBLK, 2 * H), lambda i: (i, 0)),
        out_shape=jax.ShapeDtypeStruct((N, 2 * H), jnp.float32),
    )(h, wab, bab)


_EBLK = 1920


def _msg_call(gp, w2, b2):
    return pl.pallas_call(
        _msg_body,
        grid=(GROWS // _EBLK,),
        in_specs=[
            pl.BlockSpec((_EBLK, 2 * H), lambda i: (i, 0)),
            _full((2 * H, 2 * H)),
            _full((1, 2 * H)),
        ],
        out_specs=pl.BlockSpec((_EBLK, 2 * H), lambda i: (i, 0)),
        out_shape=jax.ShapeDtypeStruct((GROWS, 2 * H), jnp.float32),
    )(gp, w2, b2)


def _bd2(w):
    k, h = w.shape
    z = jnp.zeros((2 * k, 2 * h), w.dtype)
    return z.at[:k, :h].set(w).at[k:, h:].set(w)


def _t2(b):
    return jnp.concatenate([b, b])


def kernel(x, edge_index, nW1, nb1, nW2, nb2,
           c0W1, c0b1, c0W2, c0b2,
           c1W1, c1b1, c1W2, c1b2,
           c2W1, c2b1, c2W2, c2b2):
    src = edge_index[0]
    dst = edge_index[1]

    layers = [(c0W1, c0b1, c0W2, c0b2), (c1W1, c1b1, c1W2, c1b2), (c2W1, c2b1, c2W2, c2b2)]

    wabs, babs, w2s, b2s = [], [], [], []
    for (W1, b1, W2, b2) in layers:
        Wa = W1[:H] - W1[H:]
        Wb = W1[H:]
        wabs.append(jnp.concatenate([Wa, Wb], axis=1))
        babs.append(jnp.concatenate([b1, jnp.zeros((H,), jnp.float32)])[None])
        w2s.append(_bd2(W2))
        b2s.append(_t2(b2)[None])

    lists, cnts, tots = _bin_kernel(dst, src)

    tab = _enc_call(x, nW1, nb1[None], nW2, nb2[None], wabs[0], babs[0])

    h = None
    for l in range(3):
        g = _gather_kernel(tab, lists, cnts, tots)
        mp = _msg_call(g, w2s[l], b2s[l])
        outp = _scatter_kernel(mp, lists, cnts, tots)
        h = outp[:N // 2].reshape(N, H)
        if l < 2:
            tab = _ab_call(h, wabs[l + 1], babs[l + 1])
    return h


# speedup vs baseline: 1.2649x; 1.2649x over previous
import functools

import jax
import jax.numpy as jnp
from jax import lax
from jax.experimental import pallas as pl
from jax.experimental.pallas import tpu as pltpu
from jax.experimental.pallas import tpu_sc as plsc

N = 10000
E = 320000
D_IN = 128
H = 64

NC = 2
NS = 16
NW = NC * NS
R = 320
NPAD = NW * R
SEG = 40000
NSEG = E // SEG
CAP = SEG + 16
CH = 128
GB = CH // 2
LTOT = NW * NSEG * SEG + 8 * CH
GCAP_CH = 2800
DUMP_CH = GCAP_CH - 2
GROWS = GCAP_CH * GB

_mesh = plsc.VectorSubcoreMesh(core_axis_name="c", subcore_axis_name="s")
_sc_params = pltpu.CompilerParams(needs_layout_passes=False)


def _wid():
    return lax.axis_index("s") * NC + lax.axis_index("c")


@functools.partial(
    pl.kernel,
    out_type=(
        jax.ShapeDtypeStruct((LTOT,), jnp.int32),
        jax.ShapeDtypeStruct((NW * NSEG * 16,), jnp.int32),
        jax.ShapeDtypeStruct((NW * 16,), jnp.int32),
    ),
    mesh=_mesh,
    compiler_params=_sc_params,
    scratch_types=[
        pltpu.VMEM((SEG,), jnp.int32),
        pltpu.VMEM((SEG,), jnp.int32),
        pltpu.VMEM((CAP,), jnp.int32),
        pltpu.VMEM((NSEG * 16,), jnp.int32),
        pltpu.VMEM((16,), jnp.int32),
    ],
)
def _bin_kernel(dst_hbm, src_hbm, lists_hbm, cnts_hbm, tots_hbm,
                dbuf, sbuf, cbuf, cntv, tbuf):
    t = _wid()
    lo = t * R

    def zero_body(i, c):
        cbuf[pl.ds(i * 16, 16)] = jnp.zeros((16,), jnp.int32)
        return c

    lax.fori_loop(0, CAP // 16, zero_body, 0)

    def seg_body(s, tot):
        pltpu.sync_copy(dst_hbm.at[pl.ds(s * SEG, SEG)], dbuf)
        pltpu.sync_copy(src_hbm.at[pl.ds(s * SEG, SEG)], sbuf)

        def inner(i, cur):
            d = dbuf[pl.ds(i * 16, 16)]
            sv = sbuf[pl.ds(i * 16, 16)]
            dl = d - lo
            m = (dl >= 0) & (dl < R)
            packed = (sv << 9) | (dl & 511)
            csum = plsc.cumsum(m.astype(jnp.int32))
            plsc.store_scatter(cbuf, [cur + csum - 1], packed, mask=m)
            return cur + csum[15]

        cur = lax.fori_loop(0, SEG // 16, inner, 0)
        cntv[pl.ds(s * 16, 16)] = jnp.broadcast_to(cur, (16,))
        pltpu.sync_copy(cbuf.at[pl.ds(0, SEG)],
                        lists_hbm.at[pl.ds((t * NSEG + s) * SEG, SEG)])
        return tot + (cur + CH - 1) // CH

    tot = lax.fori_loop(0, NSEG, seg_body, 0)
    tbuf[pl.ds(0, 16)] = jnp.broadcast_to(tot, (16,))
    pltpu.sync_copy(cntv, cnts_hbm.at[pl.ds(t * NSEG * 16, NSEG * 16)])
    pltpu.sync_copy(tbuf, tots_hbm.at[pl.ds(t * 16, 16)])


def _chunk_base(t, totv):
    def pre(u, eb):
        return eb + totv[pl.ds(u * 16, 16)][0]

    return lax.fori_loop(0, t, pre, 0)


@functools.partial(
    pl.kernel,
    out_type=jax.ShapeDtypeStruct((GROWS, 2 * H), jnp.float32),
    mesh=_mesh,
    compiler_params=_sc_params,
    scratch_types=[
        pltpu.VMEM((CH + 16,), jnp.int32),
        pltpu.VMEM((CH + 16,), jnp.int32),
        pltpu.VMEM((CH,), jnp.int32),
        pltpu.VMEM((CH,), jnp.int32),
        pltpu.VMEM((CH,), jnp.int32),
        pltpu.VMEM((CH,), jnp.int32),
        pltpu.VMEM((CH, 2 * H), jnp.float32),
        pltpu.VMEM((CH, 2 * H), jnp.float32),
        pltpu.VMEM((CH, 2 * H), jnp.float32),
        pltpu.VMEM((CH, 2 * H), jnp.float32),
        pltpu.VMEM((GB, 2 * H), jnp.float32),
        pltpu.VMEM((NSEG * 16,), jnp.int32),
        pltpu.VMEM((NW * 16,), jnp.int32),
        pltpu.SemaphoreType.DMA,
        pltpu.SemaphoreType.DMA,
        pltpu.SemaphoreType.DMA,
        pltpu.SemaphoreType.DMA,
    ],
)
def _gather_kernel(tab_hbm, lists_hbm, cnts_hbm, tots_hbm, g_hbm,
                   lb0, lb1, di0, si0, di1, si1, rd0, rs0, rd1, rs1,
                   gbuf, cntv, totv, sl0, sl1, sg0, sg1):
    t = _wid()
    lo = t * R
    iota = lax.iota(jnp.int32, 16)
    pltpu.sync_copy(cnts_hbm.at[pl.ds(t * NSEG * 16, NSEG * 16)], cntv)
    pltpu.sync_copy(tots_hbm, totv)
    ob0 = _chunk_base(t, totv)

    def start_list(loff, lbx, slx):
        pltpu.async_copy(lists_hbm.at[pl.ds(loff, CH)], lbx.at[pl.ds(0, CH)], slx)

    def wait_list(lbx, slx):
        pltpu.make_async_copy(lists_hbm.at[pl.ds(0, CH)], lbx.at[pl.ds(0, CH)], slx).wait()

    def unpack(lbx, dix, six, ne):
        def up(k, u):
            v = lbx[pl.ds(k * 16, 16)]
            pos = k * 16 + iota
            valid = pos < ne
            six[pl.ds(k * 16, 16)] = jnp.where(
                valid, lax.shift_right_logical(v, 9), pos)
            dix[pl.ds(k * 16, 16)] = jnp.where(valid, lo + (v & 511), pos)
            return u

        lax.fori_loop(0, CH // 16, up, 0)

    def start_rows(dix, six, rdx, rsx, sgx):
        pltpu.async_copy(tab_hbm.at[dix], rdx, sgx)
        pltpu.async_copy(tab_hbm.at[six], rsx, sgx)

    def wait_rows(dix, six, rdx, rsx, sgx):
        pltpu.make_async_copy(tab_hbm.at[dix], rdx, sgx).wait()
        pltpu.make_async_copy(tab_hbm.at[six], rsx, sgx).wait()

    def adds_and_write(rdx, rsx, ob_w):
        def add_body(q, c):
            for half in range(2):
                r = 2 * q + half
                for k in range(H // 16):
                    a = rdx[r, pl.ds(k * 16, 16)]
                    b = rsx[r, pl.ds(H + k * 16, 16)]
                    gbuf[q, pl.ds(half * H + k * 16, 16)] = a + b
            return c

        lax.fori_loop(0, GB, add_body, 0)
        pltpu.sync_copy(gbuf,
                        g_hbm.at[pl.ds(pl.multiple_of(ob_w * GB, 8), GB)])

    def seg_body(s, ob):
        cnt = cntv[pl.ds(s * 16, 16)][0]
        nch = (cnt + CH - 1) // CH
        lbase = (t * NSEG + s) * SEG

        def obw(g):
            return jnp.where(g < nch, ob + g, DUMP_CH)

        pltpu.sync_copy(lists_hbm.at[pl.ds(lbase, CH)], lb0.at[pl.ds(0, CH)])
        unpack(lb0, di0, si0, cnt)
        start_rows(di0, si0, rd0, rs0, sg0)
        start_list(lbase + CH, lb1, sl1)

        def body(i, c):
            g0 = 2 * i
            wait_list(lb1, sl1)
            unpack(lb1, di1, si1, cnt - (g0 + 1) * CH)
            start_rows(di1, si1, rd1, rs1, sg1)
            start_list(lbase + (g0 + 2) * CH, lb0, sl0)
            wait_rows(di0, si0, rd0, rs0, sg0)
            adds_and_write(rd0, rs0, obw(g0))
            wait_list(lb0, sl0)
            unpack(lb0, di0, si0, cnt - (g0 + 2) * CH)
            start_rows(di0, si0, rd0, rs0, sg0)
            start_list(lbase + (g0 + 3) * CH, lb1, sl1)
            wait_rows(di1, si1, rd1, rs1, sg1)
            adds_and_write(rd1, rs1, obw(g0 + 1))
            return c

        lax.fori_loop(0, (nch + 1) // 2, body, 0)
        wait_rows(di0, si0, rd0, rs0, sg0)
        wait_list(lb1, sl1)
        return ob + nch

    lax.fori_loop(0, NSEG, seg_body, ob0)


@functools.partial(
    pl.kernel,
    out_type=jax.ShapeDtypeStruct((NPAD // 2, 2 * H), jnp.float32),
    mesh=_mesh,
    compiler_params=_sc_params,
    scratch_types=[
        pltpu.VMEM((R // 2, 2 * H), jnp.float32),
        pltpu.VMEM((CH + 16,), jnp.int32),
        pltpu.VMEM((CH + 16,), jnp.int32),
        pltpu.VMEM((GB, 2 * H), jnp.float32),
        pltpu.VMEM((GB, 2 * H), jnp.float32),
        pltpu.VMEM((NSEG * 16,), jnp.int32),
        pltpu.VMEM((NW * 16,), jnp.int32),
        pltpu.SemaphoreType.DMA,
        pltpu.SemaphoreType.DMA,
        pltpu.SemaphoreType.DMA,
        pltpu.SemaphoreType.DMA,
    ],
)
def _scatter_kernel(m_hbm, lists_hbm, cnts_hbm, tots_hbm, out_hbm,
                    acc, lb0, lb1, rb0, rb1, cntv, totv, sl0, sl1, sm0, sm1):
    t = _wid()
    pltpu.sync_copy(cnts_hbm.at[pl.ds(t * NSEG * 16, NSEG * 16)], cntv)
    pltpu.sync_copy(tots_hbm, totv)
    ob0 = _chunk_base(t, totv)

    def zb(r, c):
        for k in range(2 * H // 16):
            acc[r, pl.ds(k * 16, 16)] = jnp.zeros((16,), jnp.float32)
        return c

    lax.fori_loop(0, R // 2, zb, 0)

    def start_pair(loff, ob, lbx, rbx, slx, smx):
        pltpu.async_copy(lists_hbm.at[pl.ds(loff, CH)], lbx.at[pl.ds(0, CH)], slx)
        pltpu.async_copy(
            m_hbm.at[pl.ds(pl.multiple_of(ob * GB, 8), GB)], rbx, smx)

    def wait_pair(lbx, rbx, slx, smx):
        pltpu.make_async_copy(lists_hbm.at[pl.ds(0, CH)], lbx.at[pl.ds(0, CH)], slx).wait()
        pltpu.make_async_copy(m_hbm.at[pl.ds(0, GB)], rbx, smx).wait()

    def consume(lbx, rbx, ne):
        def e_body(e, ec):
            p = lbx[pl.ds(e, 16)][0]
            dl = lax.bitwise_and(p, 511)
            ao = lax.bitwise_and(dl, 1) * H
            ar = lax.shift_right_logical(dl, 1)
            mo = lax.bitwise_and(e, 1) * H
            er = lax.shift_right_logical(e, 1)
            for k in range(H // 16):
                a = acc[ar, pl.ds(ao + k * 16, 16)]
                r = rbx[er, pl.ds(mo + k * 16, 16)]
                acc[ar, pl.ds(ao + k * 16, 16)] = jnp.maximum(a, r)
            return ec

        lax.fori_loop(0, jnp.maximum(ne, 0), e_body, 0)

    def seg_body(s, ob):
        cnt = cntv[pl.ds(s * 16, 16)][0]
        nch = (cnt + CH - 1) // CH
        lbase = (t * NSEG + s) * SEG

        start_pair(lbase, ob, lb0, rb0, sl0, sm0)
        start_pair(lbase + CH, ob + 1, lb1, rb1, sl1, sm1)

        def body(i, c):
            g0 = 2 * i
            wait_pair(lb0, rb0, sl0, sm0)
            consume(lb0, rb0, jnp.minimum(CH, cnt - g0 * CH))
            start_pair(lbase + (g0 + 2) * CH, ob + g0 + 2, lb0, rb0, sl0, sm0)
            wait_pair(lb1, rb1, sl1, sm1)
            consume(lb1, rb1, jnp.minimum(CH, cnt - (g0 + 1) * CH))
            start_pair(lbase + (g0 + 3) * CH, ob + g0 + 3, lb1, rb1, sl1, sm1)
            return c

        lax.fori_loop(0, (nch + 1) // 2, body, 0)
        wait_pair(lb0, rb0, sl0, sm0)
        wait_pair(lb1, rb1, sl1, sm1)
        return ob + nch

    lax.fori_loop(0, NSEG, seg_body, ob0)
    pltpu.sync_copy(acc, out_hbm.at[pl.ds(pl.multiple_of(t * (R // 2), 8), R // 2)])


def _enc_body(x_ref, w1_ref, b1_ref, w2_ref, b2_ref, wab_ref, bab_ref, tab_ref):
    x = x_ref[...]
    h = jnp.maximum(jnp.dot(x, w1_ref[...], preferred_element_type=jnp.float32) + b1_ref[...], 0.0)
    h = jnp.dot(h, w2_ref[...], preferred_element_type=jnp.float32) + b2_ref[...]
    tab_ref[...] = jnp.dot(h, wab_ref[...], preferred_element_type=jnp.float32) + bab_ref[...]


def _ab_body(h_ref, wab_ref, bab_ref, tab_ref):
    tab_ref[...] = jnp.dot(h_ref[...], wab_ref[...], preferred_element_type=jnp.float32) + bab_ref[...]


def _msg_body(gp_ref, w2_ref, b2_ref, out_ref):
    g = jnp.maximum(gp_ref[...], 0.0)
    out_ref[...] = jnp.dot(g, w2_ref[...], preferred_element_type=jnp.float32) + b2_ref[...]


def _full(shape):
    return pl.BlockSpec(shape, lambda i: (0, 0))


_NBLK = 2000


def _enc_call(x, w1, b1, w2, b2, wab, bab):
    return pl.pallas_call(
        _enc_body,
        grid=(N // _NBLK,),
        in_specs=[
            pl.BlockSpec((_NBLK, D_IN), lambda i: (i, 0)),
            _full((D_IN, H)),
            _full((1, H)),
            _full((H, H)),
            _full((1, H)),
            _full((H, 2 * H)),
            _full((1, 2 * H)),
        ],
        out_specs=pl.BlockSpec((_NBLK, 2 * H), lambda i: (i, 0)),
        out_shape=jax.ShapeDtypeStruct((N, 2 * H), jnp.float32),
    )(x, w1, b1, w2, b2, wab, bab)


def _ab_call(h, wab, bab):
    return pl.pallas_call(
        _ab_body,
        grid=(N // _NBLK,),
        in_specs=[
            pl.BlockSpec((_NBLK, H), lambda i: (i, 0)),
            _full((H, 2 * H)),
            _full((1, 2 * H)),
        ],
        out_specs=pl.BlockSpec((_NBLK, 2 * H), lambda i: (i, 0)),
        out_shape=jax.ShapeDtypeStruct((N, 2 * H), jnp.float32),
    )(h, wab, bab)


_EBLK = 1792


def _msg_call(gp, w2, b2):
    return pl.pallas_call(
        _msg_body,
        grid=(GROWS // _EBLK,),
        in_specs=[
            pl.BlockSpec((_EBLK, 2 * H), lambda i: (i, 0)),
            _full((2 * H, 2 * H)),
            _full((1, 2 * H)),
        ],
        out_specs=pl.BlockSpec((_EBLK, 2 * H), lambda i: (i, 0)),
        out_shape=jax.ShapeDtypeStruct((GROWS, 2 * H), jnp.float32),
    )(gp, w2, b2)


def _bd2(w):
    k, h = w.shape
    z = jnp.zeros((2 * k, 2 * h), w.dtype)
    return z.at[:k, :h].set(w).at[k:, h:].set(w)


def _t2(b):
    return jnp.concatenate([b, b])


def kernel(x, edge_index, nW1, nb1, nW2, nb2,
           c0W1, c0b1, c0W2, c0b2,
           c1W1, c1b1, c1W2, c1b2,
           c2W1, c2b1, c2W2, c2b2):
    src = edge_index[0]
    dst = edge_index[1]

    layers = [(c0W1, c0b1, c0W2, c0b2), (c1W1, c1b1, c1W2, c1b2), (c2W1, c2b1, c2W2, c2b2)]

    wabs, babs, w2s, b2s = [], [], [], []
    for (W1, b1, W2, b2) in layers:
        Wa = W1[:H] - W1[H:]
        Wb = W1[H:]
        wabs.append(jnp.concatenate([Wa, Wb], axis=1))
        babs.append(jnp.concatenate([b1, jnp.zeros((H,), jnp.float32)])[None])
        w2s.append(_bd2(W2))
        b2s.append(_t2(b2)[None])

    lists, cnts, tots = _bin_kernel(dst, src)

    tab = _enc_call(x, nW1, nb1[None], nW2, nb2[None], wabs[0], babs[0])

    h = None
    for l in range(3):
        g = _gather_kernel(tab, lists, cnts, tots)
        mp = _msg_call(g, w2s[l], b2s[l])
        outp = _scatter_kernel(mp, lists, cnts, tots)
        h = outp[:N // 2].reshape(N, H)
        if l < 2:
            tab = _ab_call(h, wabs[l + 1], babs[l + 1])
    return h


# speedup vs baseline: 1.3218x; 1.0450x over previous
import functools

import jax
import jax.numpy as jnp
from jax import lax
from jax.experimental import pallas as pl
from jax.experimental.pallas import tpu as pltpu
from jax.experimental.pallas import tpu_sc as plsc

N = 10000
E = 320000
D_IN = 128
H = 64

NC = 2
NS = 16
NW = NC * NS
R = 320
NPAD = NW * R
SEG = 40000
NSEG = E // SEG
CAP = SEG + 16
CH = 128
GB = CH // 2
LTOT = NW * NSEG * SEG + 8 * CH
GCAP_CH = 2800
DUMP_CH = GCAP_CH - 2
GROWS = GCAP_CH * GB

_mesh = plsc.VectorSubcoreMesh(core_axis_name="c", subcore_axis_name="s")
_sc_params = pltpu.CompilerParams(needs_layout_passes=False)


def _wid():
    return lax.axis_index("s") * NC + lax.axis_index("c")


@functools.partial(
    pl.kernel,
    out_type=(
        jax.ShapeDtypeStruct((LTOT,), jnp.int32),
        jax.ShapeDtypeStruct((NW * NSEG * 16,), jnp.int32),
        jax.ShapeDtypeStruct((NW * 16,), jnp.int32),
    ),
    mesh=_mesh,
    compiler_params=_sc_params,
    scratch_types=[
        pltpu.VMEM((SEG,), jnp.int32),
        pltpu.VMEM((SEG,), jnp.int32),
        pltpu.VMEM((CAP,), jnp.int32),
        pltpu.VMEM((NSEG * 16,), jnp.int32),
        pltpu.VMEM((16,), jnp.int32),
    ],
)
def _bin_kernel(dst_hbm, src_hbm, lists_hbm, cnts_hbm, tots_hbm,
                dbuf, sbuf, cbuf, cntv, tbuf):
    t = _wid()
    lo = t * R

    def zero_body(i, c):
        cbuf[pl.ds(i * 16, 16)] = jnp.zeros((16,), jnp.int32)
        return c

    lax.fori_loop(0, CAP // 16, zero_body, 0)

    def seg_body(s, tot):
        pltpu.sync_copy(dst_hbm.at[pl.ds(s * SEG, SEG)], dbuf)
        pltpu.sync_copy(src_hbm.at[pl.ds(s * SEG, SEG)], sbuf)

        def inner(i, cur):
            d = dbuf[pl.ds(i * 16, 16)]
            sv = sbuf[pl.ds(i * 16, 16)]
            dl = d - lo
            m = (dl >= 0) & (dl < R)
            packed = (sv << 9) | (dl & 511)
            csum = plsc.cumsum(m.astype(jnp.int32))
            plsc.store_scatter(cbuf, [cur + csum - 1], packed, mask=m)
            return cur + csum[15]

        cur = lax.fori_loop(0, SEG // 16, inner, 0)
        cntv[pl.ds(s * 16, 16)] = jnp.broadcast_to(cur, (16,))
        pltpu.sync_copy(cbuf.at[pl.ds(0, SEG)],
                        lists_hbm.at[pl.ds((t * NSEG + s) * SEG, SEG)])
        return tot + (cur + CH - 1) // CH

    tot = lax.fori_loop(0, NSEG, seg_body, 0)
    tbuf[pl.ds(0, 16)] = jnp.broadcast_to(tot, (16,))
    pltpu.sync_copy(cntv, cnts_hbm.at[pl.ds(t * NSEG * 16, NSEG * 16)])
    pltpu.sync_copy(tbuf, tots_hbm.at[pl.ds(t * 16, 16)])


def _chunk_base(t, totv):
    def pre(u, eb):
        return eb + totv[pl.ds(u * 16, 16)][0]

    return lax.fori_loop(0, t, pre, 0)


@functools.partial(
    pl.kernel,
    out_type=jax.ShapeDtypeStruct((GROWS, 2 * H), jnp.float32),
    mesh=_mesh,
    compiler_params=_sc_params,
    scratch_types=[
        pltpu.VMEM((CH + 16,), jnp.int32),
        pltpu.VMEM((CH + 16,), jnp.int32),
        pltpu.VMEM((CH,), jnp.int32),
        pltpu.VMEM((CH,), jnp.int32),
        pltpu.VMEM((CH,), jnp.int32),
        pltpu.VMEM((CH,), jnp.int32),
        pltpu.VMEM((CH, 2 * H), jnp.float32),
        pltpu.VMEM((CH, 2 * H), jnp.float32),
        pltpu.VMEM((CH, 2 * H), jnp.float32),
        pltpu.VMEM((CH, 2 * H), jnp.float32),
        pltpu.VMEM((GB, 2 * H), jnp.float32),
        pltpu.VMEM((GB, 2 * H), jnp.float32),
        pltpu.VMEM((NSEG * 16,), jnp.int32),
        pltpu.VMEM((NW * 16,), jnp.int32),
        pltpu.SemaphoreType.DMA,
        pltpu.SemaphoreType.DMA,
        pltpu.SemaphoreType.DMA,
        pltpu.SemaphoreType.DMA,
        pltpu.SemaphoreType.DMA,
        pltpu.SemaphoreType.DMA,
    ],
)
def _gather_kernel(tab_hbm, lists_hbm, cnts_hbm, tots_hbm, g_hbm,
                   lb0, lb1, di0, si0, di1, si1, rd0, rs0, rd1, rs1,
                   gb0, gb1, cntv, totv, sl0, sl1, sg0, sg1, sw0, sw1):
    t = _wid()
    lo = t * R
    iota = lax.iota(jnp.int32, 16)
    pltpu.sync_copy(cnts_hbm.at[pl.ds(t * NSEG * 16, NSEG * 16)], cntv)
    pltpu.sync_copy(tots_hbm, totv)
    ob0 = _chunk_base(t, totv)

    def start_list(loff, lbx, slx):
        pltpu.async_copy(lists_hbm.at[pl.ds(loff, CH)], lbx.at[pl.ds(0, CH)], slx)

    def wait_list(lbx, slx):
        pltpu.make_async_copy(lists_hbm.at[pl.ds(0, CH)], lbx.at[pl.ds(0, CH)], slx).wait()

    def unpack(lbx, dix, six, ne):
        def up(k, u):
            v = lbx[pl.ds(k * 16, 16)]
            pos = k * 16 + iota
            valid = pos < ne
            six[pl.ds(k * 16, 16)] = jnp.where(
                valid, lax.shift_right_logical(v, 9), pos)
            dix[pl.ds(k * 16, 16)] = jnp.where(valid, lo + (v & 511), pos)
            return u

        lax.fori_loop(0, CH // 16, up, 0)

    def start_rows(dix, six, rdx, rsx, sgx):
        pltpu.async_copy(tab_hbm.at[dix], rdx, sgx)
        pltpu.async_copy(tab_hbm.at[six], rsx, sgx)

    def wait_rows(dix, six, rdx, rsx, sgx):
        pltpu.make_async_copy(tab_hbm.at[dix], rdx, sgx).wait()
        pltpu.make_async_copy(tab_hbm.at[six], rsx, sgx).wait()

    def wait_write(gbx, swx):
        pltpu.make_async_copy(gbx, g_hbm.at[pl.ds(0, GB)], swx).wait()

    def adds_and_write(rdx, rsx, gbx, swx, ob_w):
        wait_write(gbx, swx)

        def add_body(q, c):
            for half in range(2):
                r = 2 * q + half
                for k in range(H // 16):
                    a = rdx[r, pl.ds(k * 16, 16)]
                    b = rsx[r, pl.ds(H + k * 16, 16)]
                    gbx[q, pl.ds(half * H + k * 16, 16)] = a + b
            return c

        lax.fori_loop(0, GB, add_body, 0)
        pltpu.async_copy(gbx,
                         g_hbm.at[pl.ds(pl.multiple_of(ob_w * GB, 8), GB)], swx)

    def seg_body(s, ob):
        cnt = cntv[pl.ds(s * 16, 16)][0]
        nch = (cnt + CH - 1) // CH
        lbase = (t * NSEG + s) * SEG

        def obw(g):
            return jnp.where(g < nch, ob + g, DUMP_CH)

        pltpu.sync_copy(lists_hbm.at[pl.ds(lbase, CH)], lb0.at[pl.ds(0, CH)])
        unpack(lb0, di0, si0, cnt)
        start_rows(di0, si0, rd0, rs0, sg0)
        start_list(lbase + CH, lb1, sl1)
        pltpu.async_copy(gb0, g_hbm.at[pl.ds(DUMP_CH * GB, GB)], sw0)
        pltpu.async_copy(gb1, g_hbm.at[pl.ds(DUMP_CH * GB, GB)], sw1)

        def body(i, c):
            g0 = 2 * i
            wait_list(lb1, sl1)
            unpack(lb1, di1, si1, cnt - (g0 + 1) * CH)
            start_rows(di1, si1, rd1, rs1, sg1)
            start_list(lbase + (g0 + 2) * CH, lb0, sl0)
            wait_rows(di0, si0, rd0, rs0, sg0)
            adds_and_write(rd0, rs0, gb0, sw0, obw(g0))
            wait_list(lb0, sl0)
            unpack(lb0, di0, si0, cnt - (g0 + 2) * CH)
            start_rows(di0, si0, rd0, rs0, sg0)
            start_list(lbase + (g0 + 3) * CH, lb1, sl1)
            wait_rows(di1, si1, rd1, rs1, sg1)
            adds_and_write(rd1, rs1, gb1, sw1, obw(g0 + 1))
            return c

        lax.fori_loop(0, (nch + 1) // 2, body, 0)
        wait_rows(di0, si0, rd0, rs0, sg0)
        wait_list(lb1, sl1)
        wait_write(gb0, sw0)
        wait_write(gb1, sw1)
        return ob + nch

    lax.fori_loop(0, NSEG, seg_body, ob0)


@functools.partial(
    pl.kernel,
    out_type=jax.ShapeDtypeStruct((NPAD // 2, 2 * H), jnp.float32),
    mesh=_mesh,
    compiler_params=_sc_params,
    scratch_types=[
        pltpu.VMEM((R // 2 + 1, 2 * H), jnp.float32),
        pltpu.VMEM((CH + 16,), jnp.int32),
        pltpu.VMEM((CH + 16,), jnp.int32),
        pltpu.VMEM((GB, 2 * H), jnp.float32),
        pltpu.VMEM((GB, 2 * H), jnp.float32),
        pltpu.VMEM((NSEG * 16,), jnp.int32),
        pltpu.VMEM((NW * 16,), jnp.int32),
        pltpu.SemaphoreType.DMA,
        pltpu.SemaphoreType.DMA,
        pltpu.SemaphoreType.DMA,
        pltpu.SemaphoreType.DMA,
    ],
)
def _scatter_kernel(m_hbm, lists_hbm, cnts_hbm, tots_hbm, out_hbm,
                    acc, lb0, lb1, rb0, rb1, cntv, totv, sl0, sl1, sm0, sm1):
    t = _wid()
    pltpu.sync_copy(cnts_hbm.at[pl.ds(t * NSEG * 16, NSEG * 16)], cntv)
    pltpu.sync_copy(tots_hbm, totv)
    ob0 = _chunk_base(t, totv)

    def zb(r, c):
        for k in range(2 * H // 16):
            acc[r, pl.ds(k * 16, 16)] = jnp.zeros((16,), jnp.float32)
        return c

    lax.fori_loop(0, R // 2 + 1, zb, 0)

    def start_pair(loff, ob, lbx, rbx, slx, smx):
        pltpu.async_copy(lists_hbm.at[pl.ds(loff, CH)], lbx.at[pl.ds(0, CH)], slx)
        pltpu.async_copy(
            m_hbm.at[pl.ds(pl.multiple_of(ob * GB, 8), GB)], rbx, smx)

    def wait_pair(lbx, rbx, slx, smx):
        pltpu.make_async_copy(lists_hbm.at[pl.ds(0, CH)], lbx.at[pl.ds(0, CH)], slx).wait()
        pltpu.make_async_copy(m_hbm.at[pl.ds(0, GB)], rbx, smx).wait()

    def consume(lbx, rbx, ne):
        def pair_body(i, ec):
            e0 = 2 * i
            p0 = lbx[pl.ds(e0, 16)][0]
            p1 = lbx[pl.ds(e0 + 1, 16)][0]
            dl0 = lax.bitwise_and(p0, 511)
            dl1 = lax.bitwise_and(p1, 511)
            ar0 = lax.shift_right_logical(dl0, 1)
            ar1 = jnp.where(e0 + 1 < ne, lax.shift_right_logical(dl1, 1), R // 2)
            ao0 = lax.bitwise_and(dl0, 1) * H
            ao1 = lax.bitwise_and(dl1, 1) * H
            for k in range(H // 16):
                a0 = acc[ar0, pl.ds(ao0 + k * 16, 16)]
                r0 = rbx[i, pl.ds(k * 16, 16)]
                acc[ar0, pl.ds(ao0 + k * 16, 16)] = jnp.maximum(a0, r0)
            for k in range(H // 16):
                a1 = acc[ar1, pl.ds(ao1 + k * 16, 16)]
                r1 = rbx[i, pl.ds(H + k * 16, 16)]
                acc[ar1, pl.ds(ao1 + k * 16, 16)] = jnp.maximum(a1, r1)
            return ec

        lax.fori_loop(0, jnp.maximum((ne + 1) // 2, 0), pair_body, 0)

    def seg_body(s, ob):
        cnt = cntv[pl.ds(s * 16, 16)][0]
        nch = (cnt + CH - 1) // CH
        lbase = (t * NSEG + s) * SEG

        start_pair(lbase, ob, lb0, rb0, sl0, sm0)
        start_pair(lbase + CH, ob + 1, lb1, rb1, sl1, sm1)

        def body(i, c):
            g0 = 2 * i
            wait_pair(lb0, rb0, sl0, sm0)
            consume(lb0, rb0, jnp.minimum(CH, cnt - g0 * CH))
            start_pair(lbase + (g0 + 2) * CH, ob + g0 + 2, lb0, rb0, sl0, sm0)
            wait_pair(lb1, rb1, sl1, sm1)
            consume(lb1, rb1, jnp.minimum(CH, cnt - (g0 + 1) * CH))
            start_pair(lbase + (g0 + 3) * CH, ob + g0 + 3, lb1, rb1, sl1, sm1)
            return c

        lax.fori_loop(0, (nch + 1) // 2, body, 0)
        wait_pair(lb0, rb0, sl0, sm0)
        wait_pair(lb1, rb1, sl1, sm1)
        return ob + nch

    lax.fori_loop(0, NSEG, seg_body, ob0)
    pltpu.sync_copy(acc.at[pl.ds(0, R // 2)],
                    out_hbm.at[pl.ds(pl.multiple_of(t * (R // 2), 8), R // 2)])


def _enc_body(x_ref, w1_ref, b1_ref, w2_ref, b2_ref, wab_ref, bab_ref, tab_ref):
    x = x_ref[...]
    h = jnp.maximum(jnp.dot(x, w1_ref[...], preferred_element_type=jnp.float32) + b1_ref[...], 0.0)
    h = jnp.dot(h, w2_ref[...], preferred_element_type=jnp.float32) + b2_ref[...]
    tab_ref[...] = jnp.dot(h, wab_ref[...], preferred_element_type=jnp.float32) + bab_ref[...]


def _ab_body(h_ref, wab_ref, bab_ref, tab_ref):
    tab_ref[...] = jnp.dot(h_ref[...], wab_ref[...], preferred_element_type=jnp.float32) + bab_ref[...]


def _msg_body(gp_ref, w2_ref, b2_ref, out_ref):
    g = jnp.maximum(gp_ref[...], 0.0)
    out_ref[...] = jnp.dot(g, w2_ref[...], preferred_element_type=jnp.float32) + b2_ref[...]


def _full(shape):
    return pl.BlockSpec(shape, lambda i: (0, 0))


_NBLK = 2000


def _enc_call(x, w1, b1, w2, b2, wab, bab):
    return pl.pallas_call(
        _enc_body,
        grid=(N // _NBLK,),
        in_specs=[
            pl.BlockSpec((_NBLK, D_IN), lambda i: (i, 0)),
            _full((D_IN, H)),
            _full((1, H)),
            _full((H, H)),
            _full((1, H)),
            _full((H, 2 * H)),
            _full((1, 2 * H)),
        ],
        out_specs=pl.BlockSpec((_NBLK, 2 * H), lambda i: (i, 0)),
        out_shape=jax.ShapeDtypeStruct((N, 2 * H), jnp.float32),
    )(x, w1, b1, w2, b2, wab, bab)


def _ab_call(h, wab, bab):
    return pl.pallas_call(
        _ab_body,
        grid=(N // _NBLK,),
        in_specs=[
            pl.BlockSpec((_NBLK, H), lambda i: (i, 0)),
            _full((H, 2 * H)),
            _full((1, 2 * H)),
        ],
        out_specs=pl.BlockSpec((_NBLK, 2 * H), lambda i: (i, 0)),
        out_shape=jax.ShapeDtypeStruct((N, 2 * H), jnp.float32),
    )(h, wab, bab)


_EBLK = 1792


def _msg_call(gp, w2, b2):
    return pl.pallas_call(
        _msg_body,
        grid=(GROWS // _EBLK,),
        in_specs=[
            pl.BlockSpec((_EBLK, 2 * H), lambda i: (i, 0)),
            _full((2 * H, 2 * H)),
            _full((1, 2 * H)),
        ],
        out_specs=pl.BlockSpec((_EBLK, 2 * H), lambda i: (i, 0)),
        out_shape=jax.ShapeDtypeStruct((GROWS, 2 * H), jnp.float32),
    )(gp, w2, b2)


def _bd2(w):
    k, h = w.shape
    z = jnp.zeros((2 * k, 2 * h), w.dtype)
    return z.at[:k, :h].set(w).at[k:, h:].set(w)


def _t2(b):
    return jnp.concatenate([b, b])


def kernel(x, edge_index, nW1, nb1, nW2, nb2,
           c0W1, c0b1, c0W2, c0b2,
           c1W1, c1b1, c1W2, c1b2,
           c2W1, c2b1, c2W2, c2b2):
    src = edge_index[0]
    dst = edge_index[1]

    layers = [(c0W1, c0b1, c0W2, c0b2), (c1W1, c1b1, c1W2, c1b2), (c2W1, c2b1, c2W2, c2b2)]

    wabs, babs, w2s, b2s = [], [], [], []
    for (W1, b1, W2, b2) in layers:
        Wa = W1[:H] - W1[H:]
        Wb = W1[H:]
        wabs.append(jnp.concatenate([Wa, Wb], axis=1))
        babs.append(jnp.concatenate([b1, jnp.zeros((H,), jnp.float32)])[None])
        w2s.append(_bd2(W2))
        b2s.append(_t2(b2)[None])

    lists, cnts, tots = _bin_kernel(dst, src)

    tab = _enc_call(x, nW1, nb1[None], nW2, nb2[None], wabs[0], babs[0])

    h = None
    for l in range(3):
        g = _gather_kernel(tab, lists, cnts, tots)
        mp = _msg_call(g, w2s[l], b2s[l])
        outp = _scatter_kernel(mp, lists, cnts, tots)
        h = outp[:N // 2].reshape(N, H)
        if l < 2:
            tab = _ab_call(h, wabs[l + 1], babs[l + 1])
    return h
